# baseline (speedup 1.0000x reference)
import dataclasses
import functools

import jax
import jax.numpy as jnp
from jax import lax
from jax.experimental import pallas as pl
from jax.experimental.pallas import tpu as pltpu
from jax.experimental.pallas import tpu_sc as plsc

_NC = 2
_NS = 16


def _argmax_body(s_ref, a_ref, out_ref):
    v = s_ref[...]
    e = v.shape[1]
    m = jnp.max(v, axis=1, keepdims=True)
    io = lax.broadcasted_iota(jnp.int32, v.shape, 1)
    idx = jnp.min(jnp.where(v == m, io, e), axis=1)
    x = idx[0]
    y = idx[1]
    av = a_ref[0, 0, :]
    out_ref[0, 0, :] = (x << 12) + (av << 10) + y


def _tc_argmax_phys(st, a3, bb, nblk):
    two, e, batch = st.shape
    return pl.pallas_call(
        _argmax_body,
        grid=(nblk,),
        in_specs=[
            pl.BlockSpec((2, e, bb), lambda i: (0, 0, i)),
            pl.BlockSpec((1, 1, bb), lambda i: (i, 0, 0)),
        ],
        out_specs=pl.BlockSpec((1, 1, bb), lambda i: (i, 0, 0)),
        out_shape=jax.ShapeDtypeStruct((nblk, 1, bb), jnp.int32),
        compiler_params=pltpu.CompilerParams(
            skip_device_barrier=True,
            dimension_semantics=("parallel",),
        ),
    )(st, a3)


def _sc_cp():
    cp = pltpu.CompilerParams(skip_device_barrier=True)
    if "needs_layout_passes" in pltpu.CompilerParams.__dataclass_fields__:
        cp = dataclasses.replace(cp, needs_layout_passes=False)
    return cp


_CHUNKS = {0: ((0, 256), (256, 248)), 1: ((504, 248), (752, 248))}


def _sc_argmax_slice(st_hbm, half, lb, c, sbuf_a, sbuf_b, ms, ii, semi):
    b0 = st_hbm.shape[2] - 2048 + lb * 128
    for kk in range(8):
        ms[pl.ds(16 * kk, 16)] = jnp.full((16,), -jnp.inf, jnp.float32)
        ii[pl.ds(16 * kk, 16)] = jnp.zeros((16,), jnp.int32)

    for h, chunks in _CHUNKS.items():
        @pl.when(half == h)
        def _(chunks=chunks):
            (e0a, la), (e0b, lbn) = chunks
            pltpu.async_copy(
                st_hbm.at[c, pl.ds(e0a, la), pl.ds(b0, 128)],
                sbuf_a.at[pl.ds(0, la)], semi)
            pltpu.async_copy(
                st_hbm.at[c, pl.ds(e0b, lbn), pl.ds(b0, 128)],
                sbuf_b.at[pl.ds(0, lbn)], semi)
            for buf, e0, ln in ((sbuf_a, e0a, la), (sbuf_b, e0b, lbn)):
                pltpu.make_async_copy(
                    st_hbm.at[c, pl.ds(e0, ln), pl.ds(b0, 128)],
                    buf.at[pl.ds(0, ln)], semi).wait()

                @pl.loop(0, ln // 8)
                def _(gr, buf=buf, e0=e0):
                    for kk in range(8):
                        sl = pl.ds(16 * kk, 16)
                        m = ms[sl]
                        iv = ii[sl]
                        for r in range(8):
                            row = 8 * gr + r
                            v = buf[row, sl]
                            ev = jnp.full((16,), 1, jnp.int32) * (e0 + row)
                            gt = v > m
                            m = jnp.where(gt, v, m)
                            iv = jnp.where(gt, ev, iv)
                        ms[sl] = m
                        ii[sl] = iv


def _sc_table_repack(t2, st):
    nw = _NC * _NS
    ex, na, ey = t2.shape
    mesh = plsc.VectorSubcoreMesh(core_axis_name="c", subcore_axis_name="s")
    starts = list(range(0, ey - 16, 16)) + [ey - 16]

    ng = (ex + nw - 1) // nw

    @functools.partial(
        pl.kernel,
        out_type=[
            jax.ShapeDtypeStruct((ex * 4096,), jnp.float32),
            jax.ShapeDtypeStruct((8192,), jnp.float32),
            jax.ShapeDtypeStruct((8192,), jnp.int32),
        ],
        mesh=mesh,
        compiler_params=_sc_cp(),
        scratch_types=[
            pltpu.VMEM((na, ey), jnp.float32),
            pltpu.VMEM((na, ey), jnp.float32),
            pltpu.VMEM((4096,), jnp.float32),
            pltpu.VMEM((4096,), jnp.float32),
            pltpu.VMEM((256, 128), jnp.float32),
            pltpu.VMEM((256, 128), jnp.float32),
            pltpu.VMEM((128,), jnp.float32),
            pltpu.VMEM((128,), jnp.int32),
            pltpu.SemaphoreType.DMA,
            pltpu.SemaphoreType.DMA,
        ],
    )
    def k(t_hbm, st_hbm, o_hbm, pm_hbm, pi_hbm, buf_a, buf_b, out_a, out_b,
          sbuf_a, sbuf_b, ms, ii, semi, semo):
        wid = lax.axis_index("s") * _NC + lax.axis_index("c")
        half = wid & 1
        lb = wid >> 1

        def fire_in(g, buf):
            x = wid + nw * g

            @pl.when(x < ex)
            def _():
                pltpu.async_copy(t_hbm.at[x], buf, semi)

        def step(g, buf, ob):
            x = wid + nw * g

            @pl.when(x < ex)
            def _():
                @pl.when(g >= 2)
                def _():
                    pltpu.make_async_copy(
                        ob, o_hbm.at[pl.ds(0, 4096)], semo).wait()
                pltpu.make_async_copy(t_hbm.at[x], buf, semi).wait()
                for j in range(na):
                    for c in starts:
                        ob[pl.ds(j * 1024 + c, 16)] = buf[j, pl.ds(c, 16)]
                pltpu.async_copy(
                    ob, o_hbm.at[pl.ds(x * 4096, 4096)], semo)
                fire_in(g + 2, buf)

        fire_in(0, buf_a)
        fire_in(1, buf_b)

        @pl.loop(0, ng // 2)
        def _(i):
            step(2 * i, buf_a, out_a)
            step(2 * i + 1, buf_b, out_b)

        pltpu.make_async_copy(out_a, o_hbm.at[pl.ds(0, 4096)], semo).wait()
        pltpu.make_async_copy(out_b, o_hbm.at[pl.ds(0, 4096)], semo).wait()

        for c in range(2):
            _sc_argmax_slice(st_hbm, half, lb, c, sbuf_a, sbuf_b, ms, ii,
                             semi)
            base = c * 4096 + half * 2048
            pltpu.sync_copy(ms, pm_hbm.at[pl.ds(base + lb * 128, 128)])
            pltpu.sync_copy(ii, pi_hbm.at[pl.ds(base + lb * 128, 128)])

    return k(t2, st)


def _sc_table_gather(tlin, ptc2, pm, pi, ai):
    ntc = ptc2.shape[0]
    nw = _NC * _NS
    rpw = 4
    wtc = ntc // rpw
    batch = ai.shape[0]
    mesh = plsc.VectorSubcoreMesh(core_axis_name="c", subcore_axis_name="s")

    @functools.partial(
        pl.kernel,
        out_type=jax.ShapeDtypeStruct((batch // 128, 128), jnp.float32),
        mesh=mesh,
        compiler_params=_sc_cp(),
        scratch_types=[
            pltpu.VMEM((rpw, 128), jnp.int32),
            pltpu.VMEM((rpw, 128), jnp.float32),
            pltpu.VMEM((512,), jnp.float32),
            pltpu.VMEM((512,), jnp.float32),
            pltpu.VMEM((512,), jnp.float32),
            pltpu.VMEM((512,), jnp.float32),
            pltpu.VMEM((512,), jnp.int32),
            pltpu.VMEM((512,), jnp.int32),
            pltpu.VMEM((512,), jnp.int32),
            pltpu.VMEM((512,), jnp.int32),
            pltpu.VMEM((512,), jnp.int32),
            pltpu.SemaphoreType.DMA,
        ],
    )
    def k(t_hbm, p_hbm, pm_hbm, pi_hbm, a_hbm, o_hbm, iv, vv,
          mx0, mx1, my0, my1, x0, x1, y0, y1, ab, sem):
        wid = lax.axis_index("s") * _NC + lax.axis_index("c")

        @pl.when(wid < wtc)
        def _():
            pltpu.sync_copy(p_hbm.at[pl.ds(rpw * wid, rpw)], iv)

        @pl.when(wid >= wtc)
        def _():
            t = wid - wtc
            e0 = 512 * t
            pltpu.sync_copy(pm_hbm.at[pl.ds(e0, 512)], mx0)
            pltpu.sync_copy(pm_hbm.at[pl.ds(2048 + e0, 512)], mx1)
            pltpu.sync_copy(pm_hbm.at[pl.ds(4096 + e0, 512)], my0)
            pltpu.sync_copy(pm_hbm.at[pl.ds(6144 + e0, 512)], my1)
            pltpu.sync_copy(pi_hbm.at[pl.ds(e0, 512)], x0)
            pltpu.sync_copy(pi_hbm.at[pl.ds(2048 + e0, 512)], x1)
            pltpu.sync_copy(pi_hbm.at[pl.ds(4096 + e0, 512)], y0)
            pltpu.sync_copy(pi_hbm.at[pl.ds(6144 + e0, 512)], y1)
            pltpu.sync_copy(
                a_hbm.at[pl.ds(batch - 2048 + e0, 512)], ab)
            for ch in range(32):
                sl = pl.ds(16 * ch, 16)
                x = jnp.where(mx1[sl] > mx0[sl], x1[sl], x0[sl])
                y = jnp.where(my1[sl] > my0[sl], y1[sl], y0[sl])
                p = (x << 12) + (ab[sl] << 10) + y
                iv.at[ch // 8][pl.ds((ch % 8) * 16, 16)] = p

        copies = [
            pltpu.async_copy(t_hbm.at[iv.at[j]], vv.at[j], sem)
            for j in range(rpw)
        ]
        for c in copies:
            c.wait()
        pltpu.sync_copy(vv, o_hbm.at[pl.ds(rpw * wid, rpw)])

    return k(tlin, ptc2, pm, pi, ai)


def kernel(s, a, env_size, table):
    batch = s.shape[0]
    b_tc = batch - 2048
    st = jnp.transpose(s, (1, 2, 0))
    ai = a.astype(jnp.int32)
    a3 = ai.reshape(-1, 1, 2048)
    p_tc = _tc_argmax_phys(st, a3, bb=2048, nblk=b_tc // 2048)
    ptc2 = p_tc.reshape(b_tc // 128, 128)
    t2 = jnp.transpose(table, (0, 2, 1))
    tlin, pm, pi = _sc_table_repack(t2, st)
    out2 = _sc_table_gather(tlin, ptc2, pm, pi, ai)
    return out2.reshape(batch)


# speedup vs baseline: 1.2455x; 1.2455x over previous
import dataclasses
import functools

import jax
import jax.numpy as jnp
from jax import lax
from jax.experimental import pallas as pl
from jax.experimental.pallas import tpu as pltpu
from jax.experimental.pallas import tpu_sc as plsc

_NC = 2
_NS = 16


def _argmax_body(s_ref, a_ref, out_ref):
    v = s_ref[...]
    e = v.shape[1]
    m = jnp.max(v, axis=1, keepdims=True)
    io = lax.broadcasted_iota(jnp.int32, v.shape, 1)
    idx = jnp.min(jnp.where(v == m, io, e), axis=1)
    x = idx[0]
    y = idx[1]
    av = a_ref[0, 0, :]
    out_ref[0, 0, :] = (x << 12) + (av << 10) + y


def _tc_argmax_phys(st, a3, bb):
    two, e, batch = st.shape
    nblk = batch // bb
    return pl.pallas_call(
        _argmax_body,
        grid=(nblk,),
        in_specs=[
            pl.BlockSpec((2, e, bb), lambda i: (0, 0, i)),
            pl.BlockSpec((1, 1, bb), lambda i: (i, 0, 0)),
        ],
        out_specs=pl.BlockSpec((1, 1, bb), lambda i: (i, 0, 0)),
        out_shape=jax.ShapeDtypeStruct((nblk, 1, bb), jnp.int32),
        compiler_params=pltpu.CompilerParams(
            skip_device_barrier=True,
            dimension_semantics=("parallel",),
        ),
    )(st, a3)


def _sc_cp():
    cp = pltpu.CompilerParams(skip_device_barrier=True)
    if "needs_layout_passes" in pltpu.CompilerParams.__dataclass_fields__:
        cp = dataclasses.replace(cp, needs_layout_passes=False)
    return cp


def _sc_table_repack(t2):
    nw = _NC * _NS
    ex, na, ey = t2.shape
    mesh = plsc.VectorSubcoreMesh(core_axis_name="c", subcore_axis_name="s")
    starts = list(range(0, ey - 16, 16)) + [ey - 16]

    ng = (ex + nw - 1) // nw

    @functools.partial(
        pl.kernel,
        out_type=jax.ShapeDtypeStruct((ex * 4096,), jnp.float32),
        mesh=mesh,
        compiler_params=_sc_cp(),
        scratch_types=[
            pltpu.VMEM((na, ey), jnp.float32),
            pltpu.VMEM((na, ey), jnp.float32),
            pltpu.VMEM((4096,), jnp.float32),
            pltpu.VMEM((4096,), jnp.float32),
            pltpu.SemaphoreType.DMA,
            pltpu.SemaphoreType.DMA,
        ],
    )
    def k(t_hbm, o_hbm, buf_a, buf_b, out_a, out_b, semi, semo):
        wid = lax.axis_index("s") * _NC + lax.axis_index("c")

        def fire_in(g, buf):
            x = wid + nw * g

            @pl.when(x < ex)
            def _():
                pltpu.async_copy(t_hbm.at[x], buf, semi)

        def step(g, buf, ob):
            x = wid + nw * g

            @pl.when(x < ex)
            def _():
                @pl.when(g >= 2)
                def _():
                    pltpu.make_async_copy(
                        ob, o_hbm.at[pl.ds(0, 4096)], semo).wait()
                pltpu.make_async_copy(t_hbm.at[x], buf, semi).wait()
                for j in range(na):
                    for c in starts:
                        ob[pl.ds(j * 1024 + c, 16)] = buf[j, pl.ds(c, 16)]
                pltpu.async_copy(
                    ob, o_hbm.at[pl.ds(x * 4096, 4096)], semo)
                fire_in(g + 2, buf)

        fire_in(0, buf_a)
        fire_in(1, buf_b)

        @pl.loop(0, ng // 2)
        def _(i):
            step(2 * i, buf_a, out_a)
            step(2 * i + 1, buf_b, out_b)

        pltpu.make_async_copy(out_a, o_hbm.at[pl.ds(0, 4096)], semo).wait()
        pltpu.make_async_copy(out_b, o_hbm.at[pl.ds(0, 4096)], semo).wait()

    return k(t2)


def _sc_table_gather(tlin, p2):
    nrows = p2.shape[0]
    nw = _NC * _NS
    rpw = nrows // nw
    mesh = plsc.VectorSubcoreMesh(core_axis_name="c", subcore_axis_name="s")

    @functools.partial(
        pl.kernel,
        out_type=jax.ShapeDtypeStruct((nrows, 128), jnp.float32),
        mesh=mesh,
        compiler_params=_sc_cp(),
        scratch_types=[
            pltpu.VMEM((rpw, 128), jnp.int32),
            pltpu.VMEM((rpw, 128), jnp.float32),
            pltpu.SemaphoreType.DMA,
        ],
    )
    def k(t_hbm, p_hbm, o_hbm, iv, vv, sem):
        wid = lax.axis_index("s") * _NC + lax.axis_index("c")
        pltpu.sync_copy(p_hbm.at[pl.ds(rpw * wid, rpw)], iv)
        copies = [
            pltpu.async_copy(t_hbm.at[iv.at[j]], vv.at[j], sem)
            for j in range(rpw)
        ]
        for c in copies:
            c.wait()
        pltpu.sync_copy(vv, o_hbm.at[pl.ds(rpw * wid, rpw)])

    return k(tlin, p2)


def kernel(s, a, env_size, table):
    batch = s.shape[0]
    st = jnp.transpose(s, (1, 2, 0))
    a3 = a.astype(jnp.int32).reshape(-1, 1, 1024)
    p = _tc_argmax_phys(st, a3, bb=1024)
    p2 = p.reshape(batch // 128, 128)
    t2 = jnp.transpose(table, (0, 2, 1))
    tlin = _sc_table_repack(t2)
    out2 = _sc_table_gather(tlin, p2)
    return out2.reshape(batch)
